# baseline (speedup 1.0000x reference)
import functools
import numpy as np
import jax
import jax.numpy as jnp
from jax import lax
from jax.experimental import pallas as pl
from jax.experimental.pallas import tpu as pltpu

_NUM_EMBEDS = 16


def _pool_grid(num_embeds):
    if num_embeds in (1, 2, 3, 5, 7):
        return (num_embeds, 1)
    return {4: (2, 2), 6: (3, 2), 8: (4, 2), 9: (3, 3),
            16: (4, 4), 25: (5, 5), 36: (6, 6)}[num_embeds]


def _pool_matrix(H, W, gh, gw):
    P = np.zeros((gh * gw, H * W), dtype=np.float32)
    for i in range(gh):
        h0 = (i * H) // gh
        h1 = -(-((i + 1) * H) // gh)
        for j in range(gw):
            w0 = (j * W) // gw
            w1 = -(-((j + 1) * W) // gw)
            cnt = float((h1 - h0) * (w1 - w0))
            for hh in range(h0, h1):
                for ww in range(w0, w1):
                    P[i * gw + j, hh * W + ww] = 1.0 / cnt
    return P


def _fused_kernel(tb, first_steps, x_ref, pb_hbm, w_hbm, b_hbm, o_ref,
                  pb_buf, w_buf, b_buf, sems):
    i = pl.program_id(0)
    is_first = i == first_steps[0]
    for s in first_steps[1:]:
        is_first = jnp.logical_or(is_first, i == s)

    @pl.when(is_first)
    def _():
        c0 = pltpu.make_async_copy(pb_hbm, pb_buf, sems.at[0])
        c1 = pltpu.make_async_copy(w_hbm, w_buf, sems.at[1])
        c2 = pltpu.make_async_copy(b_hbm, b_buf, sems.at[2])
        c0.start()
        c1.start()
        c2.start()
        c0.wait()
        c1.wait()
        c2.wait()

    tb_hw = pb_buf.shape[1]
    c = x_ref.shape[2]
    xflat = x_ref[...].reshape(tb_hw, c)
    pooled = lax.dot_general(
        pb_buf[...], xflat,
        dimension_numbers=(((1,), (0,)), ((), ())),
        preferred_element_type=jnp.float32)
    y = lax.dot_general(
        pooled, w_buf[...],
        dimension_numbers=(((1,), (1,)), ((), ())),
        preferred_element_type=jnp.float32)
    y = (y + b_buf[...]).astype(o_ref.dtype)
    o_ref[...] = y.reshape(o_ref.shape)


def kernel(x, weight, bias):
    B, C, H, W = x.shape
    N = weight.shape[0]
    P = _NUM_EMBEDS
    gh, gw = _pool_grid(P)
    HW = H * W

    tb = 8 if B % 8 == 0 else 1
    grid_b = B // tb

    pmat = _pool_matrix(H, W, gh, gw)
    pblk_np = np.zeros((tb * P, tb * HW), np.float32)
    for b in range(tb):
        pblk_np[b * P:(b + 1) * P, b * HW:(b + 1) * HW] = pmat
    pblk = jnp.asarray(pblk_np)

    x_hwc = jnp.transpose(x, (0, 2, 3, 1)).reshape(B, HW, C)
    b2 = bias.reshape(1, N)

    cost = pl.CostEstimate(
        flops=2 * B * (P * HW * C + P * C * N),
        transcendentals=0,
        bytes_accessed=4 * (B * C * HW + N * C + N + B * P * N),
    )

    first_steps = (0, grid_b // 2) if grid_b % 2 == 0 else (0,)

    out = pl.pallas_call(
        functools.partial(_fused_kernel, tb, first_steps),
        out_shape=jax.ShapeDtypeStruct((B, P, N), x.dtype),
        grid=(grid_b,),
        in_specs=[
            pl.BlockSpec((tb, HW, C), lambda i: (i, 0, 0)),
            pl.BlockSpec(memory_space=pl.ANY),
            pl.BlockSpec(memory_space=pl.ANY),
            pl.BlockSpec(memory_space=pl.ANY),
        ],
        out_specs=pl.BlockSpec((tb, P, N), lambda i: (i, 0, 0)),
        scratch_shapes=[
            pltpu.VMEM((tb * P, tb * HW), jnp.float32),
            pltpu.VMEM((N, C), jnp.float32),
            pltpu.VMEM((1, N), jnp.float32),
            pltpu.SemaphoreType.DMA((3,)),
        ],
        compiler_params=pltpu.CompilerParams(
            dimension_semantics=("parallel",),
            vmem_limit_bytes=64 * 1024 * 1024,
        ),
        cost_estimate=cost,
    )(x_hwc, pblk, weight, b2)

    return out


# speedup vs baseline: 1.1130x; 1.1130x over previous
import functools
import numpy as np
import jax
import jax.numpy as jnp
from jax import lax
from jax.experimental import pallas as pl
from jax.experimental.pallas import tpu as pltpu

_NUM_EMBEDS = 16


def _pool_grid(num_embeds):
    if num_embeds in (1, 2, 3, 5, 7):
        return (num_embeds, 1)
    return {4: (2, 2), 6: (3, 2), 8: (4, 2), 9: (3, 3),
            16: (4, 4), 25: (5, 5), 36: (6, 6)}[num_embeds]


def _pool_matrix(H, W, gh, gw):
    P = np.zeros((gh * gw, H * W), dtype=np.float32)
    for i in range(gh):
        h0 = (i * H) // gh
        h1 = -(-((i + 1) * H) // gh)
        for j in range(gw):
            w0 = (j * W) // gw
            w1 = -(-((j + 1) * W) // gw)
            cnt = float((h1 - h0) * (w1 - w0))
            for hh in range(h0, h1):
                for ww in range(w0, w1):
                    P[i * gw + j, hh * W + ww] = 1.0 / cnt
    return P


def _fused_kernel(tb, x_ref, pb_ref, w_ref, b_ref, o_ref):
    half_hw = pb_ref.shape[1]
    c = x_ref.shape[2]
    hw = x_ref.shape[1]
    xflat = x_ref[...].reshape(tb * hw, c)
    pb = pb_ref[...]
    halves = [
        lax.dot_general(
            pb, xflat[h * half_hw:(h + 1) * half_hw, :],
            dimension_numbers=(((1,), (0,)), ((), ())),
            preferred_element_type=jnp.float32)
        for h in range(tb * hw // half_hw)
    ]
    pooled = jnp.concatenate(halves, axis=0) if len(halves) > 1 else halves[0]
    y = lax.dot_general(
        pooled, w_ref[...],
        dimension_numbers=(((1,), (1,)), ((), ())),
        preferred_element_type=jnp.float32)
    y = (y + b_ref[...]).astype(o_ref.dtype)
    o_ref[...] = y.reshape(o_ref.shape)


def kernel(x, weight, bias):
    B, C, H, W = x.shape
    N = weight.shape[0]
    P = _NUM_EMBEDS
    gh, gw = _pool_grid(P)
    HW = H * W

    tb = 8 if B % 8 == 0 else 1
    grid_b = B // tb

    pmat = _pool_matrix(H, W, gh, gw)
    half = tb // 2 if tb % 2 == 0 else tb
    pblk_np = np.zeros((half * P, half * HW), np.float32)
    for b in range(half):
        pblk_np[b * P:(b + 1) * P, b * HW:(b + 1) * HW] = pmat
    pblk = jnp.asarray(pblk_np)

    x_hwc = jnp.transpose(x, (0, 2, 3, 1)).reshape(B, HW, C)
    b2 = bias.reshape(1, N)

    cost = pl.CostEstimate(
        flops=2 * B * (P * HW * C + P * C * N),
        transcendentals=0,
        bytes_accessed=4 * (B * C * HW + N * C + N + B * P * N),
    )

    out = pl.pallas_call(
        functools.partial(_fused_kernel, tb),
        out_shape=jax.ShapeDtypeStruct((B, P, N), x.dtype),
        grid=(grid_b,),
        in_specs=[
            pl.BlockSpec((tb, HW, C), lambda i: (i, 0, 0)),
            pl.BlockSpec((half * P, half * HW), lambda i: (0, 0)),
            pl.BlockSpec((N, C), lambda i: (0, 0)),
            pl.BlockSpec((1, N), lambda i: (0, 0)),
        ],
        out_specs=pl.BlockSpec((tb, P, N), lambda i: (i, 0, 0)),
        compiler_params=pltpu.CompilerParams(
            dimension_semantics=("parallel",),
            vmem_limit_bytes=64 * 1024 * 1024,
        ),
        cost_estimate=cost,
    )(x_hwc, pblk, weight, b2)

    return out


# speedup vs baseline: 1.1339x; 1.0188x over previous
import functools
import numpy as np
import jax
import jax.numpy as jnp
from jax import lax
from jax.experimental import pallas as pl
from jax.experimental.pallas import tpu as pltpu

_NUM_EMBEDS = 16


def _pool_grid(num_embeds):
    if num_embeds in (1, 2, 3, 5, 7):
        return (num_embeds, 1)
    return {4: (2, 2), 6: (3, 2), 8: (4, 2), 9: (3, 3),
            16: (4, 4), 25: (5, 5), 36: (6, 6)}[num_embeds]


def _pool_matrix(H, W, gh, gw):
    P = np.zeros((gh * gw, H * W), dtype=np.float32)
    for i in range(gh):
        h0 = (i * H) // gh
        h1 = -(-((i + 1) * H) // gh)
        for j in range(gw):
            w0 = (j * W) // gw
            w1 = -(-((j + 1) * W) // gw)
            cnt = float((h1 - h0) * (w1 - w0))
            for hh in range(h0, h1):
                for ww in range(w0, w1):
                    P[i * gw + j, hh * W + ww] = 1.0 / cnt
    return P


def _fused_kernel(tb, x_ref, pb_ref, w_ref, b_ref, o_ref):
    tb_hw = pb_ref.shape[1]
    c = x_ref.shape[2]
    xflat = x_ref[...].reshape(tb_hw, c)
    pooled = lax.dot_general(
        pb_ref[...], xflat,
        dimension_numbers=(((1,), (0,)), ((), ())),
        preferred_element_type=jnp.float32)
    y = lax.dot_general(
        pooled, w_ref[...],
        dimension_numbers=(((1,), (1,)), ((), ())),
        preferred_element_type=jnp.float32)
    y = (y + b_ref[...]).astype(o_ref.dtype)
    o_ref[...] = y.reshape(o_ref.shape)


def kernel(x, weight, bias):
    B, C, H, W = x.shape
    N = weight.shape[0]
    P = _NUM_EMBEDS
    gh, gw = _pool_grid(P)
    HW = H * W

    tb = 8 if B % 8 == 0 else 1
    grid_b = B // tb

    pmat = _pool_matrix(H, W, gh, gw)
    pblk_np = np.zeros((tb * P, tb * HW), np.float32)
    for b in range(tb):
        pblk_np[b * P:(b + 1) * P, b * HW:(b + 1) * HW] = pmat
    pblk = jnp.asarray(pblk_np)

    x_hwc = jnp.transpose(x, (0, 2, 3, 1)).reshape(B, HW, C)
    b2 = bias.reshape(1, N)

    cost = pl.CostEstimate(
        flops=2 * B * (P * HW * C + P * C * N),
        transcendentals=0,
        bytes_accessed=4 * (B * C * HW + N * C + N + B * P * N),
    )

    out = pl.pallas_call(
        functools.partial(_fused_kernel, tb),
        out_shape=jax.ShapeDtypeStruct((B, P, N), x.dtype),
        grid=(grid_b,),
        in_specs=[
            pl.BlockSpec((tb, HW, C), lambda i: (i, 0, 0)),
            pl.BlockSpec((tb * P, tb * HW), lambda i: (0, 0)),
            pl.BlockSpec((N, C), lambda i: (0, 0)),
            pl.BlockSpec((1, N), lambda i: (0, 0)),
        ],
        out_specs=pl.BlockSpec((tb, P, N), lambda i: (i, 0, 0)),
        compiler_params=pltpu.CompilerParams(
            dimension_semantics=("parallel",),
            vmem_limit_bytes=64 * 1024 * 1024,
        ),
        cost_estimate=cost,
    )(x_hwc, pblk, weight, b2)

    return out
